# baseline (speedup 1.0000x reference)
import jax
import jax.numpy as jnp
from jax.experimental import pallas as pl
from jax.experimental.pallas import tpu as pltpu

_G = 16


def _stream_kernel(res_ref, x_ref, ptx_hbm, xi_ref, ptx_out_hbm, dma_sem):
    i = pl.program_id(0)

    @pl.when(i == 0)
    def _start():
        pltpu.make_async_copy(ptx_hbm, ptx_out_hbm, dma_sem).start()

    xi_ref[...] = x_ref[...] + res_ref[0]

    @pl.when(i == _G - 1)
    def _finish():
        pltpu.make_async_copy(ptx_hbm, ptx_out_hbm, dma_sem).wait()


def kernel(x, ptx, bs, height, width, point_edges, point_src_dirs, point_tgt_dirs):
    C, H, W = x.shape[1], x.shape[2], x.shape[3]
    n_pts = ptx.shape[0]
    residual = (
        (jnp.asarray(bs) - 1) + (jnp.asarray(height) - H) + (jnp.asarray(width) - W)
    ).astype(x.dtype)
    res = residual.reshape(1)

    xb = C // _G

    xi, ptx_out = pl.pallas_call(
        _stream_kernel,
        grid=(_G,),
        in_specs=[
            pl.BlockSpec(memory_space=pltpu.SMEM),
            pl.BlockSpec((1, xb, H, W), lambda i: (0, i, 0, 0)),
            pl.BlockSpec(memory_space=pl.ANY),
        ],
        out_specs=[
            pl.BlockSpec((1, xb, H, W), lambda i: (0, i, 0, 0)),
            pl.BlockSpec(memory_space=pl.ANY),
        ],
        out_shape=[
            jax.ShapeDtypeStruct((1, C, H, W), x.dtype),
            jax.ShapeDtypeStruct((n_pts, C), ptx.dtype),
        ],
        scratch_shapes=[pltpu.SemaphoreType.DMA],
    )(res, x, ptx)

    return (xi, ptx_out)


# speedup vs baseline: 14.0011x; 14.0011x over previous
import jax
import jax.numpy as jnp
from jax.experimental import pallas as pl
from jax.experimental.pallas import tpu as pltpu


def _stream_kernel(res_ref, x_ref, ptx_ref, xi_ref, ptx_out_ref):
    xi_ref[...] = x_ref[...] + res_ref[0]

    @pl.when(pl.program_id(0) % 4 == 0)
    def _copy_ptx():
        ptx_out_ref[...] = ptx_ref[...]


def kernel(x, ptx, bs, height, width, point_edges, point_src_dirs, point_tgt_dirs):
    C, H, W = x.shape[1], x.shape[2], x.shape[3]
    n_pts = ptx.shape[0]
    residual = (
        (jnp.asarray(bs) - 1) + (jnp.asarray(height) - H) + (jnp.asarray(width) - W)
    ).astype(x.dtype)
    res = residual.reshape(1)

    G = 16
    xb = C // G
    pb = n_pts // 4

    xi, ptx_out = pl.pallas_call(
        _stream_kernel,
        grid=(G,),
        in_specs=[
            pl.BlockSpec(memory_space=pltpu.SMEM),
            pl.BlockSpec((1, xb, H, W), lambda i: (0, i, 0, 0)),
            pl.BlockSpec((pb, C), lambda i: (i // 4, 0)),
        ],
        out_specs=[
            pl.BlockSpec((1, xb, H, W), lambda i: (0, i, 0, 0)),
            pl.BlockSpec((pb, C), lambda i: (i // 4, 0)),
        ],
        out_shape=[
            jax.ShapeDtypeStruct((1, C, H, W), x.dtype),
            jax.ShapeDtypeStruct((n_pts, C), ptx.dtype),
        ],
    )(res, x, ptx)

    return (xi, ptx_out)


# speedup vs baseline: 14.9373x; 1.0669x over previous
import jax
import jax.numpy as jnp
from jax.experimental import pallas as pl
from jax.experimental.pallas import tpu as pltpu


def _stream_kernel(res_ref, x_ref, xi_ref):
    xi_ref[...] = x_ref[...] + res_ref[0]


def kernel(x, ptx, bs, height, width, point_edges, point_src_dirs, point_tgt_dirs):
    C, H, W = x.shape[1], x.shape[2], x.shape[3]
    residual = (
        (jnp.asarray(bs) - 1) + (jnp.asarray(height) - H) + (jnp.asarray(width) - W)
    ).astype(x.dtype)
    res = residual.reshape(1)

    G = 4
    xb = C // G

    xi = pl.pallas_call(
        _stream_kernel,
        grid=(G,),
        in_specs=[
            pl.BlockSpec(memory_space=pltpu.SMEM),
            pl.BlockSpec((1, xb, H, W), lambda i: (0, i, 0, 0)),
        ],
        out_specs=pl.BlockSpec((1, xb, H, W), lambda i: (0, i, 0, 0)),
        out_shape=jax.ShapeDtypeStruct((1, C, H, W), x.dtype),
    )(res, x)

    return (xi, ptx)
